# Initial kernel scaffold; baseline (speedup 1.0000x reference)
import jax
import jax.numpy as jnp
from jax.experimental import pallas as pl


def kernel(features, edge_index, Wself0, Wneigh0, b0, Wself1, Wneigh1, b1, Wself2, Wneigh2, b2):
    raise NotImplementedError("write your pallas kernel here")


# speedup vs baseline: 6.6527x; 6.6527x over previous
import functools

import jax
import jax.numpy as jnp
from jax import lax
from jax.experimental import pallas as pl
from jax.experimental.pallas import tpu as pltpu
from jax.experimental.pallas import tpu_sc as plsc

N = 10000
E = 320000
D = 128
NC = 2
NS = 16
NW = NC * NS

CHUNK = 128
NCHUNKS = E // CHUNK
CPW = (NCHUNKS + NW - 1) // NW
DEG_SLICE = 640
N_DEG = DEG_SLICE * NS
ROWS_TILE = 632
ROWS_LAST = N - ROWS_TILE * (NS - 1)


def _make_sc_agg(with_deg):
    if with_deg:
        out_type = (jax.ShapeDtypeStruct((NC, N, D), jnp.float32),
                    jax.ShapeDtypeStruct((NC, 1, N_DEG), jnp.float32))
    else:
        out_type = jax.ShapeDtypeStruct((NC, N, D), jnp.float32)

    scratch = [
        pltpu.VMEM((CHUNK, D), jnp.float32),
        pltpu.VMEM((CHUNK, D), jnp.float32),
        pltpu.VMEM((CHUNK,), jnp.int32),
        pltpu.VMEM((CHUNK,), jnp.int32),
        pltpu.VMEM_SHARED((N, D), jnp.float32),
        pltpu.SemaphoreType.DMA,
    ]
    if with_deg:
        scratch += [
            pltpu.VMEM((CHUNK,), jnp.float32),
            pltpu.VMEM((DEG_SLICE,), jnp.float32),
            pltpu.VMEM_SHARED((N_DEG,), jnp.float32),
        ]

    mesh = plsc.VectorSubcoreMesh(core_axis_name="c", subcore_axis_name="s")

    @functools.partial(pl.kernel, out_type=out_type, mesh=mesh,
                       scratch_types=scratch)
    def sc_agg(*refs):
        if with_deg:
            (h_hbm, src_hbm, dst_hbm, agg_hbm, deg_hbm,
             rowbuf, zbuf, srcbuf, dstbuf, agg_sh, sem,
             onesbuf, zdbuf, deg_sh) = refs
        else:
            (h_hbm, src_hbm, dst_hbm, agg_hbm,
             rowbuf, zbuf, srcbuf, dstbuf, agg_sh, sem) = refs

        c = lax.axis_index("c")
        s = lax.axis_index("s")
        wid = s * NC + c

        zv = jnp.zeros((16,), jnp.float32)

        def zrow(r, carry):
            for j in range(D // 16):
                zbuf[r, pl.ds(j * 16, 16)] = zv
            return carry

        lax.fori_loop(0, CHUNK, zrow, 0)

        if with_deg:
            ov = jnp.ones((16,), jnp.float32)
            for j in range(CHUNK // 16):
                onesbuf[pl.ds(j * 16, 16)] = ov
            for j in range(DEG_SLICE // 16):
                zdbuf[pl.ds(j * 16, 16)] = zv

        def _zero_rows(base, nrows):
            full, rem = nrows // CHUNK, nrows % CHUNK
            for k in range(full):
                pltpu.sync_copy(zbuf, agg_sh.at[pl.ds(base + k * CHUNK, CHUNK), :])
            if rem:
                pltpu.sync_copy(zbuf.at[pl.ds(0, rem), :],
                                agg_sh.at[pl.ds(base + full * CHUNK, rem), :])

        @pl.when(s < NS - 1)
        def _():
            _zero_rows(s * ROWS_TILE, ROWS_TILE)

        @pl.when(s == NS - 1)
        def _():
            _zero_rows((NS - 1) * ROWS_TILE, ROWS_LAST)

        if with_deg:
            pltpu.sync_copy(zdbuf, deg_sh.at[pl.ds(s * DEG_SLICE, DEG_SLICE)])

        plsc.subcore_barrier()

        def echunk(j, carry):
            ch = wid + NW * j

            @pl.when(ch < NCHUNKS)
            def _():
                off = ch * CHUNK
                pltpu.sync_copy(src_hbm.at[pl.ds(off, CHUNK)], srcbuf)
                pltpu.sync_copy(dst_hbm.at[pl.ds(off, CHUNK)], dstbuf)
                pltpu.async_copy(h_hbm.at[srcbuf], rowbuf, sem).wait()
                pltpu.sync_copy(rowbuf, agg_sh.at[dstbuf], add=True)
                if with_deg:
                    pltpu.sync_copy(onesbuf, deg_sh.at[dstbuf], add=True)

            return carry

        lax.fori_loop(0, CPW, echunk, 0)

        plsc.subcore_barrier()

        @pl.when(s < NS - 1)
        def _():
            pltpu.sync_copy(agg_sh.at[pl.ds(s * ROWS_TILE, ROWS_TILE), :],
                            agg_hbm.at[c, pl.ds(s * ROWS_TILE, ROWS_TILE), :])

        @pl.when(s == NS - 1)
        def _():
            pltpu.sync_copy(agg_sh.at[pl.ds((NS - 1) * ROWS_TILE, ROWS_LAST), :],
                            agg_hbm.at[c, pl.ds((NS - 1) * ROWS_TILE, ROWS_LAST), :])
        if with_deg:
            pltpu.sync_copy(deg_sh.at[pl.ds(s * DEG_SLICE, DEG_SLICE)],
                            deg_hbm.at[c, 0, pl.ds(s * DEG_SLICE, DEG_SLICE)])

    return sc_agg


@functools.lru_cache(maxsize=None)
def _get_sc_agg(with_deg):
    return _make_sc_agg(with_deg)


RB = 1000
GRID = N // RB


def _combine_body(act, h_ref, agg_ref, deg_ref, ws_ref, wn_ref, b_ref, o_ref):
    deg = jnp.sum(deg_ref[...], axis=(0, 1))
    inv = (1.0 / jnp.maximum(deg, 1.0))[:, None]
    agg = agg_ref[0] + agg_ref[1]
    out = jnp.dot(h_ref[...], ws_ref[...], preferred_element_type=jnp.float32)
    out = out + jnp.dot(agg * inv, wn_ref[...], preferred_element_type=jnp.float32)
    out = out + b_ref[...]
    if act:
        out = jnp.maximum(out, 0.0)
    o_ref[...] = out


def _combine(h, agg2, deg_r, Ws, Wn, b, act):
    return pl.pallas_call(
        functools.partial(_combine_body, act),
        out_shape=jax.ShapeDtypeStruct((N, D), jnp.float32),
        grid=(GRID,),
        in_specs=[
            pl.BlockSpec((RB, D), lambda i: (i, 0)),
            pl.BlockSpec((NC, RB, D), lambda i: (0, i, 0)),
            pl.BlockSpec((1, NC, RB), lambda i: (i, 0, 0)),
            pl.BlockSpec((D, D), lambda i: (0, 0)),
            pl.BlockSpec((D, D), lambda i: (0, 0)),
            pl.BlockSpec((1, D), lambda i: (0, 0)),
        ],
        out_specs=pl.BlockSpec((RB, D), lambda i: (i, 0)),
    )(h, agg2, deg_r, Ws, Wn, b)


def kernel(features, edge_index, Wself0, Wneigh0, b0, Wself1, Wneigh1, b1,
           Wself2, Wneigh2, b2):
    src = edge_index[0]
    dst = edge_index[1]

    agg0, deg2 = _get_sc_agg(True)(features, src, dst)
    deg_r = deg2[:, 0, :N].reshape(NC, GRID, RB).transpose(1, 0, 2)

    h1 = _combine(features, agg0, deg_r, Wself0, Wneigh0, b0.reshape(1, D), True)
    agg1 = _get_sc_agg(False)(h1, src, dst)
    h2 = _combine(h1, agg1, deg_r, Wself1, Wneigh1, b1.reshape(1, D), True)
    agg2 = _get_sc_agg(False)(h2, src, dst)
    h3 = _combine(h2, agg2, deg_r, Wself2, Wneigh2, b2.reshape(1, D), False)
    return h3
